# baseline (speedup 1.0000x reference)
import functools

import jax
import jax.numpy as jnp
from jax import lax
from jax.experimental import pallas as pl
from jax.experimental.pallas import tpu as pltpu
from jax.experimental.pallas import tpu_sc as plsc


def _sc_broadcast_rows(b, s, d, dtype):
    info = plsc.get_sparse_core_info()
    nw = info.num_cores * info.num_subcores
    rows_per_w = s // nw
    chunk = rows_per_w
    while chunk * d * 4 > 128 * 1024:
        chunk //= 2
    n_chunks = rows_per_w // chunk
    mesh = plsc.VectorSubcoreMesh(core_axis_name="c", subcore_axis_name="s")

    @functools.partial(
        pl.kernel,
        mesh=mesh,
        out_type=jax.ShapeDtypeStruct((b, s, d), dtype),
        scratch_types=[
            pltpu.VMEM((n_chunks, chunk, d), dtype),
            pltpu.SemaphoreType.DMA,
            pltpu.SemaphoreType.DMA,
        ],
    )
    def k(table_hbm, out_hbm, buf, rsem, wsem):
        wid = lax.axis_index("s") * info.num_cores + lax.axis_index("c")
        base = wid * rows_per_w
        reads = []
        for i in range(n_chunks):
            r0 = base + i * chunk
            reads.append(
                pltpu.async_copy(table_hbm.at[pl.ds(r0, chunk)], buf.at[i], rsem)
            )
        writes = []
        for i in range(n_chunks):
            reads[i].wait()
            r0 = base + i * chunk
            for j in range(b):
                bb = (wid + j) % b
                writes.append(
                    pltpu.async_copy(buf.at[i], out_hbm.at[bb, pl.ds(r0, chunk)], wsem)
                )
        for w in writes:
            w.wait()

    return k


def kernel(x, pos_table):
    b, s, _ = x.shape
    d = pos_table.shape[1]
    return _sc_broadcast_rows(b, s, d, pos_table.dtype)(pos_table)


# speedup vs baseline: 1.0196x; 1.0196x over previous
import functools

import jax
import jax.numpy as jnp
from jax import lax
from jax.experimental import pallas as pl
from jax.experimental.pallas import tpu as pltpu
from jax.experimental.pallas import tpu_sc as plsc


def _sc_broadcast_rows(b, s, d, dtype):
    info = plsc.get_sparse_core_info()
    nw = info.num_cores * info.num_subcores
    rows_per_w = s // nw
    chunk = rows_per_w
    while chunk * d * 4 > 256 * 1024:
        chunk //= 2
    n_chunks = rows_per_w // chunk
    mesh = plsc.VectorSubcoreMesh(core_axis_name="c", subcore_axis_name="s")

    @functools.partial(
        pl.kernel,
        mesh=mesh,
        out_type=jax.ShapeDtypeStruct((b, s, d), dtype),
        scratch_types=[
            pltpu.VMEM((n_chunks, chunk, d), dtype),
            pltpu.SemaphoreType.DMA,
            pltpu.SemaphoreType.DMA,
        ],
    )
    def k(table_hbm, out_hbm, buf, rsem, wsem):
        wid = lax.axis_index("s") * info.num_cores + lax.axis_index("c")
        base = wid * rows_per_w
        reads = []
        for i in range(n_chunks):
            r0 = base + i * chunk
            reads.append(
                pltpu.async_copy(table_hbm.at[pl.ds(r0, chunk)], buf.at[i], rsem)
            )
        writes = []
        for i in range(n_chunks):
            reads[i].wait()
            r0 = base + i * chunk
            for j in range(b):
                bb = (wid + j) % b
                writes.append(
                    pltpu.async_copy(buf.at[i], out_hbm.at[bb, pl.ds(r0, chunk)], wsem)
                )
        for w in writes:
            w.wait()

    return k


def kernel(x, pos_table):
    b, s, _ = x.shape
    d = pos_table.shape[1]
    return _sc_broadcast_rows(b, s, d, pos_table.dtype)(pos_table)
